# baseline (device time: 88187 ns/iter reference)
import jax
import jax.numpy as jnp
from jax import lax
from jax.experimental import pallas as pl
from jax.experimental.pallas import tpu as pltpu

N_DEV = 8
B, SQ, DM = 2, 256, 768
HQ_PER = 8
DH = 64
DQ_PER = HQ_PER * DH
DKV = 2 * DH


def kernel(x, Wq, Wo, Wk, Wv):
    xb = x.astype(jnp.bfloat16)
    wqb = Wq.astype(jnp.bfloat16)
    wkb = Wk.astype(jnp.bfloat16)
    wvb = Wv.astype(jnp.bfloat16)
    wob = Wo.astype(jnp.bfloat16)

    def body(x_ref, wq_ref, wk_ref, wv_ref, wo_ref, out_ref,
             comm_ref, attn_ref, send_sems, recv_sems):
        my = lax.axis_index("i")
        left = lax.rem(my + N_DEV - 1, N_DEV)
        right = lax.rem(my + 1, N_DEV)

        barrier_sem = pltpu.get_barrier_semaphore()
        for nbr in (left, right):
            pl.semaphore_signal(
                barrier_sem, inc=1,
                device_id=(nbr,), device_id_type=pl.DeviceIdType.MESH,
            )
        pl.semaphore_wait(barrier_sem, 2)

        xm = x_ref[...].reshape(B * SQ, DM)
        q = jnp.dot(xm, wq_ref[...], preferred_element_type=jnp.float32)
        kv_start = my * DKV
        k = jnp.dot(xm, wk_ref[:, pl.ds(kv_start, DKV)],
                    preferred_element_type=jnp.float32)
        v = jnp.dot(xm, wv_ref[:, pl.ds(kv_start, DKV)],
                    preferred_element_type=jnp.float32)
        qb = q.astype(jnp.bfloat16)
        kb = k.astype(jnp.bfloat16)
        vb = v.astype(jnp.bfloat16)

        for b in range(B):
            for hh in range(HQ_PER):
                g = hh // 4
                qs = qb[b * SQ:(b + 1) * SQ, hh * DH:(hh + 1) * DH]
                ks = kb[b * SQ:(b + 1) * SQ, g * DH:(g + 1) * DH]
                vs = vb[b * SQ:(b + 1) * SQ, g * DH:(g + 1) * DH]
                s = jnp.dot(qs, ks.T, preferred_element_type=jnp.float32) * 0.125
                m = jnp.max(s, axis=-1, keepdims=True)
                p = jnp.exp(s - m)
                l = jnp.sum(p, axis=-1, keepdims=True)
                o = jnp.dot(p.astype(jnp.bfloat16), vs,
                            preferred_element_type=jnp.float32) / l
                attn_ref[b * SQ:(b + 1) * SQ, hh * DH:(hh + 1) * DH] = (
                    o.astype(jnp.bfloat16))

        partial = jnp.dot(attn_ref[...], wo_ref[...],
                          preferred_element_type=jnp.float32)

        comm_ref[0] = partial.astype(jnp.bfloat16)
        acc = partial
        for h in range(N_DEV - 1):
            send_slot = h % 2
            recv_slot = (h + 1) % 2
            rdma = pltpu.make_async_remote_copy(
                src_ref=comm_ref.at[send_slot],
                dst_ref=comm_ref.at[recv_slot],
                send_sem=send_sems.at[send_slot],
                recv_sem=recv_sems.at[recv_slot],
                device_id=(right,),
                device_id_type=pl.DeviceIdType.MESH,
            )
            rdma.start()
            rdma.wait()
            acc = acc + comm_ref[recv_slot].astype(jnp.float32)

        out_ref[...] = acc.reshape(B, SQ, DM)

    return pl.pallas_call(
        body,
        out_shape=jax.ShapeDtypeStruct((B, SQ, DM), jnp.float32),
        in_specs=[pl.BlockSpec(memory_space=pltpu.VMEM)] * 5,
        out_specs=pl.BlockSpec(memory_space=pltpu.VMEM),
        scratch_shapes=[
            pltpu.VMEM((2, B * SQ, DM), jnp.bfloat16),
            pltpu.VMEM((B * SQ, DQ_PER), jnp.bfloat16),
            pltpu.SemaphoreType.DMA((2,)),
            pltpu.SemaphoreType.DMA((2,)),
        ],
        compiler_params=pltpu.CompilerParams(collective_id=0),
    )(xb, wqb, wkb, wvb, wob)


# device time: 30824 ns/iter; 2.8610x vs baseline; 2.8610x over previous
import jax
import jax.numpy as jnp
from jax import lax
from jax.experimental import pallas as pl
from jax.experimental.pallas import tpu as pltpu

N_DEV = 8
B, SQ, DM = 2, 256, 768
HQ_PER = 8
DH = 64
DQ_PER = HQ_PER * DH
DKV = 2 * DH
ROWS = B * SQ
CHUNK = ROWS // N_DEV


def kernel(x, Wq, Wo, Wk, Wv):
    xb = x.astype(jnp.bfloat16)
    wqb = Wq.astype(jnp.bfloat16)
    wkb = Wk.astype(jnp.bfloat16)
    wvb = Wv.astype(jnp.bfloat16)
    wob = Wo.astype(jnp.bfloat16)

    def body(x_ref, wq_ref, wk_ref, wv_ref, wo_ref, out_ref,
             partial_ref, buf1, send_sems1, recv_sems1,
             send_sems2, recv_sems2):
        my = lax.axis_index("i")

        barrier_sem = pltpu.get_barrier_semaphore()
        for k in range(1, N_DEV):
            pl.semaphore_signal(
                barrier_sem, inc=1,
                device_id=(lax.rem(my + k, N_DEV),),
                device_id_type=pl.DeviceIdType.MESH,
            )
        pl.semaphore_wait(barrier_sem, N_DEV - 1)

        xm = x_ref[...].reshape(ROWS, DM)
        q = jnp.dot(xm, wq_ref[...], preferred_element_type=jnp.float32)
        kv_start = my * DKV
        k_ = jnp.dot(xm, wk_ref[:, pl.ds(kv_start, DKV)],
                     preferred_element_type=jnp.float32)
        v_ = jnp.dot(xm, wv_ref[:, pl.ds(kv_start, DKV)],
                     preferred_element_type=jnp.float32)
        qb = q.astype(jnp.bfloat16)
        kb = k_.astype(jnp.bfloat16)
        vb = v_.astype(jnp.bfloat16)

        attn_cols = []
        for b in range(B):
            row_blocks = []
            for hh in range(HQ_PER):
                g = hh // 4
                qs = qb[b * SQ:(b + 1) * SQ, hh * DH:(hh + 1) * DH]
                ks = kb[b * SQ:(b + 1) * SQ, g * DH:(g + 1) * DH]
                vs = vb[b * SQ:(b + 1) * SQ, g * DH:(g + 1) * DH]
                s = jnp.dot(qs, ks.T, preferred_element_type=jnp.float32) * 0.125
                m = jnp.max(s, axis=-1, keepdims=True)
                p = jnp.exp(s - m)
                l = jnp.sum(p, axis=-1, keepdims=True)
                o = jnp.dot(p.astype(jnp.bfloat16), vs,
                            preferred_element_type=jnp.float32) / l
                row_blocks.append(o.astype(jnp.bfloat16))
            attn_cols.append(jnp.concatenate(row_blocks, axis=1))
        attn = jnp.concatenate(attn_cols, axis=0)

        partial = jnp.dot(attn, wo_ref[...],
                          preferred_element_type=jnp.float32)
        partial_ref[...] = partial.astype(jnp.bfloat16)

        p1 = []
        for k in range(1, N_DEV):
            dst = lax.rem(my + k, N_DEV)
            rdma = pltpu.make_async_remote_copy(
                src_ref=partial_ref.at[pl.ds(dst * CHUNK, CHUNK)],
                dst_ref=buf1.at[k],
                send_sem=send_sems1.at[k],
                recv_sem=recv_sems1.at[k],
                device_id=(dst,),
                device_id_type=pl.DeviceIdType.MESH,
            )
            rdma.start()
            p1.append(rdma)
        buf1[0] = partial_ref[pl.ds(my * CHUNK, CHUNK), :]

        for rdma in p1:
            rdma.wait_recv()
        red = buf1[0].astype(jnp.float32)
        for k in range(1, N_DEV):
            red = red + buf1[k].astype(jnp.float32)
        out_ref[pl.ds(my * CHUNK, CHUNK), :] = red.astype(jnp.bfloat16)

        p2 = []
        for k in range(1, N_DEV):
            dst = lax.rem(my + k, N_DEV)
            rdma = pltpu.make_async_remote_copy(
                src_ref=out_ref.at[pl.ds(my * CHUNK, CHUNK)],
                dst_ref=out_ref.at[pl.ds(my * CHUNK, CHUNK)],
                send_sem=send_sems2.at[k],
                recv_sem=recv_sems2.at[k],
                device_id=(dst,),
                device_id_type=pl.DeviceIdType.MESH,
            )
            rdma.start()
            p2.append(rdma)

        for rdma in p2:
            rdma.wait_recv()
        for rdma in p1:
            rdma.wait_send()
        for rdma in p2:
            rdma.wait_send()

    res = pl.pallas_call(
        body,
        out_shape=jax.ShapeDtypeStruct((ROWS, DM), jnp.bfloat16),
        in_specs=[pl.BlockSpec(memory_space=pltpu.VMEM)] * 5,
        out_specs=pl.BlockSpec(memory_space=pltpu.VMEM),
        scratch_shapes=[
            pltpu.VMEM((ROWS, DM), jnp.bfloat16),
            pltpu.VMEM((N_DEV, CHUNK, DM), jnp.bfloat16),
            pltpu.SemaphoreType.DMA((N_DEV,)),
            pltpu.SemaphoreType.DMA((N_DEV,)),
            pltpu.SemaphoreType.DMA((N_DEV,)),
            pltpu.SemaphoreType.DMA((N_DEV,)),
        ],
        compiler_params=pltpu.CompilerParams(collective_id=0),
    )(xb, wqb, wkb, wvb, wob)
    return res.astype(jnp.float32).reshape(B, SQ, DM)
